# Initial kernel scaffold; baseline (speedup 1.0000x reference)
import jax
import jax.numpy as jnp
from jax.experimental import pallas as pl


def kernel(input_ids, weight):
    raise NotImplementedError("write your pallas kernel here")


# speedup vs baseline: 1.7303x; 1.7303x over previous
import functools

import jax
import jax.numpy as jnp
import numpy as np
from jax import lax
from jax.experimental import pallas as pl
from jax.experimental.pallas import tpu as pltpu
from jax.experimental.pallas import tpu_sc as plsc

VOCAB = 100000
EMBED_DIM = 2048
SCALE = float(np.sqrt(EMBED_DIM))

NUM_CORES = 2
NUM_SUBCORES = 16
NUM_WORKERS = NUM_CORES * NUM_SUBCORES

CHUNK = 8
NBUF = 4


def _sc_embedding(n_rows: int):
    n_per_w = n_rows // NUM_WORKERS
    n_chunks = n_per_w // CHUNK
    assert n_chunks % NBUF == 0

    mesh = plsc.VectorSubcoreMesh(
        core_axis_name="c", subcore_axis_name="s",
        num_cores=NUM_CORES, num_subcores=NUM_SUBCORES,
    )

    @functools.partial(
        pl.kernel,
        mesh=mesh,
        out_type=jax.ShapeDtypeStruct((n_rows, EMBED_DIM), jnp.float32),
        scratch_types=[
            pltpu.VMEM((n_per_w,), jnp.int32),
            pltpu.VMEM((NBUF * CHUNK, EMBED_DIM), jnp.float32),
            [pltpu.SemaphoreType.DMA] * NBUF,
            [pltpu.SemaphoreType.DMA] * NBUF,
        ],
    )
    def run(idx_hbm, table_hbm, out_hbm, idx_v, bufs, gsems, ssems):
        wid = lax.axis_index("s") * NUM_CORES + lax.axis_index("c")
        base = wid * n_per_w
        pltpu.sync_copy(idx_hbm.at[pl.ds(base, n_per_w)], idx_v)

        def gather(chunk, buf):
            return pltpu.make_async_copy(
                table_hbm.at[idx_v.at[pl.ds(chunk * CHUNK, CHUNK)]],
                bufs.at[pl.ds(buf * CHUNK, CHUNK)],
                gsems[buf],
            )

        def scatter(chunk, buf):
            return pltpu.make_async_copy(
                bufs.at[pl.ds(buf * CHUNK, CHUNK)],
                out_hbm.at[pl.ds(base + chunk * CHUNK, CHUNK)],
                ssems[buf],
            )

        for b in range(NBUF):
            gather(b, b).start()

        scale = jnp.float32(SCALE)

        def step(g2, _):
            for b in range(NBUF):
                g = g2 * NBUF + b
                gather(g, b).wait()

                def scale_body(j, _):
                    col = j * 16
                    for r in range(CHUNK):
                        row = b * CHUNK + r
                        v = bufs[row, pl.ds(col, 16)]
                        bufs[row, pl.ds(col, 16)] = v * scale
                    return 0

                lax.fori_loop(0, EMBED_DIM // 16, scale_body, 0)
                scatter(g, b).start()

                bp = (b + NBUF - 1) % NBUF
                gp = g - 1

                @pl.when((gp >= 0) & (gp + NBUF < n_chunks))
                def _():
                    scatter(gp, bp).wait()
                    gather(gp + NBUF, bp).start()

            return 0

        lax.fori_loop(0, n_chunks // NBUF, step, 0)

        for b in range(NBUF):
            scatter(n_chunks - NBUF + b, b).wait()

    return run


def kernel(input_ids, weight):
    b, s = input_ids.shape
    idx = input_ids.reshape(-1).astype(jnp.int32)
    out = _sc_embedding(b * s)(idx, weight)
    return out.reshape(b, s, EMBED_DIM)
